# Initial kernel scaffold; baseline (speedup 1.0000x reference)
import jax
import jax.numpy as jnp
from jax.experimental import pallas as pl


def kernel(x, edge_index, edge_weight, kernel):
    raise NotImplementedError("write your pallas kernel here")


# speedup vs baseline: 5.8056x; 5.8056x over previous
import functools

import jax
import jax.numpy as jnp
from jax import lax
from jax.experimental import pallas as pl
from jax.experimental.pallas import tpu as pltpu
from jax.experimental.pallas import tpu_sc as plsc

_N_NODES = 10000
_N_EDGES = 320000
_D = 128
_F = 16

_NC = 2
_NS = 16
_NW = _NC * _NS
_E_TILE = _N_EDGES // _NW
_CHUNK = 80
_NCHUNK = _E_TILE // _CHUNK
_N_PAD = 10240
_ROWS_TILE = _N_PAD // _NS


def _matmul_body(x_ref, w_ref, h_ref):
    h_ref[...] = jnp.dot(x_ref[...], w_ref[...],
                         preferred_element_type=jnp.float32)


def _softmax_body(p_ref, o_ref):
    s = p_ref[0] + p_ref[1]
    m = jnp.max(s, axis=-1, keepdims=True)
    e = jnp.exp(s - m)
    o_ref[...] = e / jnp.sum(e, axis=-1, keepdims=True)


def _agg_body(h_hbm, src_hbm, dst_hbm, ew_hbm, out_hbm,
              sidx, didx, wv, msgs, stage, acc, sem):
    cid = lax.axis_index("c")
    sid = lax.axis_index("s")
    wid = cid * _NS + sid

    zero = jnp.zeros((_F,), jnp.float32)

    def _zrow(r, c):
        stage[r, :] = zero
        return c

    lax.fori_loop(0, _ROWS_TILE, _zrow, 0)
    r0 = sid * _ROWS_TILE
    pltpu.sync_copy(stage, acc.at[pl.ds(r0, _ROWS_TILE)])
    plsc.subcore_barrier()

    base = wid * _E_TILE

    def _chunk(k, carry):
        off = base + k * _CHUNK
        pltpu.sync_copy(src_hbm.at[pl.ds(off, _CHUNK)], sidx)
        pltpu.sync_copy(dst_hbm.at[pl.ds(off, _CHUNK)], didx)
        pltpu.sync_copy(ew_hbm.at[pl.ds(off, _CHUNK)], wv)
        pltpu.async_copy(h_hbm.at[sidx], msgs, sem).wait()

        def _scale(g, c):
            e0 = g * 16
            w16 = wv[pl.ds(e0, 16)]
            for j in range(16):
                msgs[e0 + j, :] = msgs[e0 + j, :] * w16[j]
            return c

        lax.fori_loop(0, _CHUNK // 16, _scale, 0)
        pltpu.sync_copy(msgs, acc.at[didx], add=True)
        return carry

    lax.fori_loop(0, _NCHUNK, _chunk, 0)
    plsc.subcore_barrier()
    pltpu.sync_copy(acc.at[pl.ds(r0, _ROWS_TILE)],
                    out_hbm.at[cid, pl.ds(r0, _ROWS_TILE)])


def kernel(x, edge_index, edge_weight, kernel):
    w = kernel
    src = edge_index[0].astype(jnp.int32)
    dst = edge_index[1].astype(jnp.int32)
    ew = edge_weight.astype(jnp.float32)

    h = pl.pallas_call(
        _matmul_body,
        out_shape=jax.ShapeDtypeStruct((_N_NODES, _F), jnp.float32),
    )(x, w)

    mesh = plsc.VectorSubcoreMesh(core_axis_name="c", subcore_axis_name="s")
    agg_fn = functools.partial(
        pl.kernel,
        mesh=mesh,
        out_type=jax.ShapeDtypeStruct((_NC, _N_PAD, _F), jnp.float32),
        scratch_types=[
            pltpu.VMEM((_CHUNK,), jnp.int32),
            pltpu.VMEM((_CHUNK,), jnp.int32),
            pltpu.VMEM((_CHUNK,), jnp.float32),
            pltpu.VMEM((_CHUNK, _F), jnp.float32),
            pltpu.VMEM((_ROWS_TILE, _F), jnp.float32),
            pltpu.VMEM_SHARED((_N_PAD, _F), jnp.float32),
            pltpu.SemaphoreType.DMA,
        ],
        compiler_params=pltpu.CompilerParams(use_tc_tiling_on_sc=False),
    )(_agg_body)
    parts = agg_fn(h, src, dst, ew)[:, :_N_NODES, :]

    out = pl.pallas_call(
        _softmax_body,
        out_shape=jax.ShapeDtypeStruct((_N_NODES, _F), jnp.float32),
    )(parts)
    return out


# speedup vs baseline: 12.3541x; 2.1280x over previous
import functools

import jax
import jax.numpy as jnp
from jax import lax
from jax.experimental import pallas as pl
from jax.experimental.pallas import tpu as pltpu
from jax.experimental.pallas import tpu_sc as plsc

_N_NODES = 10000
_N_EDGES = 320000
_D = 128
_F = 16

_NC = 2
_NS = 16
_NW = _NC * _NS
_E_TILE = _N_EDGES // _NW
_CHUNK = 128
_NCHUNK = 80
_E_PAD = _CHUNK * _NCHUNK
_N_PAD = 10240
_ROWS_TILE = _N_PAD // _NS


def _matmul_body(x_ref, w_ref, h_ref):
    h_ref[...] = jnp.dot(x_ref[...], w_ref[...],
                         preferred_element_type=jnp.float32)


def _softmax_body(p_ref, o_ref):
    s = p_ref[0] + p_ref[1]
    m = jnp.max(s, axis=-1, keepdims=True)
    e = jnp.exp(s - m)
    o_ref[...] = e / jnp.sum(e, axis=-1, keepdims=True)


def _agg_body(h_hbm, src_hbm, dst_hbm, ew_hbm, zero_hbm, out_hbm,
              sidx, didx, wv, msg0, msg1, acc, gsem0, gsem1):
    cid = lax.axis_index("c")
    sid = lax.axis_index("s")
    wid = cid * _NS + sid
    r0 = sid * _ROWS_TILE

    pltpu.sync_copy(zero_hbm, acc.at[pl.ds(r0, _ROWS_TILE)])
    plsc.subcore_barrier()

    pltpu.sync_copy(src_hbm.at[wid], sidx)
    pltpu.sync_copy(dst_hbm.at[wid], didx)
    pltpu.sync_copy(ew_hbm.at[wid], wv)

    def _scale(msg, k):
        for g in range(_CHUNK // 16):
            w16 = wv[k, pl.ds(g * 16, 16)]
            for j in range(16):
                e = g * 16 + j
                msg[e, :] = msg[e, :] * w16[j]

    def _gather(k, msg, sem):
        pltpu.async_copy(h_hbm.at[sidx.at[k]], msg, sem)

    def _wait(msg, sem):
        pltpu.make_async_copy(h_hbm.at[sidx.at[0]], msg, sem).wait()

    def _scatter(msg, k):
        pltpu.sync_copy(msg, acc.at[didx.at[k]], add=True)

    _gather(0, msg0, gsem0)

    def _pair(i, carry):
        k0 = 2 * i
        k1 = k0 + 1
        _gather(k1, msg1, gsem1)
        _wait(msg0, gsem0)
        _scale(msg0, k0)
        _scatter(msg0, k0)
        _gather(k0 + 2, msg0, gsem0)
        _wait(msg1, gsem1)
        _scale(msg1, k1)
        _scatter(msg1, k1)
        return carry

    lax.fori_loop(0, _NCHUNK // 2 - 1, _pair, 0)

    _gather(_NCHUNK - 1, msg1, gsem1)
    _wait(msg0, gsem0)
    _scale(msg0, _NCHUNK - 2)
    _scatter(msg0, _NCHUNK - 2)
    _wait(msg1, gsem1)
    _scale(msg1, _NCHUNK - 1)
    _scatter(msg1, _NCHUNK - 1)

    plsc.subcore_barrier()
    pltpu.sync_copy(acc.at[pl.ds(r0, _ROWS_TILE)],
                    out_hbm.at[cid, pl.ds(r0, _ROWS_TILE)])


def kernel(x, edge_index, edge_weight, kernel):
    w = kernel
    src = edge_index[0].astype(jnp.int32)
    dst = edge_index[1].astype(jnp.int32)
    ew = edge_weight.astype(jnp.float32)

    pad = ((0, 0), (0, _E_PAD - _E_TILE))
    src_p = jnp.pad(src.reshape(_NW, _E_TILE), pad).reshape(_NW, _NCHUNK, _CHUNK)
    dst_p = jnp.pad(dst.reshape(_NW, _E_TILE), pad).reshape(_NW, _NCHUNK, _CHUNK)
    ew_p = jnp.pad(ew.reshape(_NW, _E_TILE), pad).reshape(_NW, _NCHUNK, _CHUNK)
    zero = jnp.zeros((_ROWS_TILE, _F), jnp.float32)

    h = pl.pallas_call(
        _matmul_body,
        out_shape=jax.ShapeDtypeStruct((_N_NODES, _F), jnp.float32),
    )(x, w)

    mesh = plsc.VectorSubcoreMesh(core_axis_name="c", subcore_axis_name="s")
    agg_fn = functools.partial(
        pl.kernel,
        mesh=mesh,
        out_type=jax.ShapeDtypeStruct((_NC, _N_PAD, _F), jnp.float32),
        scratch_types=[
            pltpu.VMEM((_NCHUNK, _CHUNK), jnp.int32),
            pltpu.VMEM((_NCHUNK, _CHUNK), jnp.int32),
            pltpu.VMEM((_NCHUNK, _CHUNK), jnp.float32),
            pltpu.VMEM((_CHUNK, _F), jnp.float32),
            pltpu.VMEM((_CHUNK, _F), jnp.float32),
            pltpu.VMEM_SHARED((_N_PAD, _F), jnp.float32),
            pltpu.SemaphoreType.DMA,
            pltpu.SemaphoreType.DMA,
        ],
        compiler_params=pltpu.CompilerParams(use_tc_tiling_on_sc=False),
    )(_agg_body)
    parts = agg_fn(h, src_p, dst_p, ew_p, zero)[:, :_N_NODES, :]

    out = pl.pallas_call(
        _softmax_body,
        out_shape=jax.ShapeDtypeStruct((_N_NODES, _F), jnp.float32),
    )(parts)
    return out


# speedup vs baseline: 15.1470x; 1.2261x over previous
import functools

import jax
import jax.numpy as jnp
from jax import lax
from jax.experimental import pallas as pl
from jax.experimental.pallas import tpu as pltpu
from jax.experimental.pallas import tpu_sc as plsc

_N_NODES = 10000
_N_EDGES = 320000
_D = 128
_F = 16

_NC = 2
_NS = 16
_NW = _NC * _NS
_E_TILE = _N_EDGES // _NW
_CHUNK = 80
_NCHUNK = _E_TILE // _CHUNK
_N_PAD = 10240
_ROWS_TILE = _N_PAD // _NS
_SM_BLK = 1000


def _matmul_body(x_ref, w_ref, h_ref):
    h_ref[...] = jnp.dot(x_ref[...], w_ref[...],
                         preferred_element_type=jnp.float32)


def _softmax_body(p_ref, o_ref):
    s = p_ref[0] + p_ref[1]
    m = jnp.max(s, axis=-1, keepdims=True)
    e = jnp.exp(s - m)
    o_ref[...] = e / jnp.sum(e, axis=-1, keepdims=True)


def _agg_body(h_hbm, src_hbm, dst_hbm, ew_hbm, zero_hbm, out_hbm,
              sidx, didx, wv, msg0, msg1, acc, gsem0, gsem1, ssem0, ssem1):
    cid = lax.axis_index("c")
    sid = lax.axis_index("s")
    wid = cid * _NS + sid
    r0 = sid * _ROWS_TILE
    e0 = wid * _E_TILE

    pltpu.sync_copy(zero_hbm, acc.at[pl.ds(r0, _ROWS_TILE)])
    pltpu.sync_copy(src_hbm.at[pl.ds(e0, _E_TILE)], sidx)
    pltpu.sync_copy(ew_hbm.at[pl.ds(e0, _E_TILE)], wv)
    pltpu.sync_copy(dst_hbm.at[wid], didx)
    plsc.subcore_barrier()

    def _scale(msg, k):
        for g in range(_CHUNK // 16):
            w16 = wv[pl.ds(k * _CHUNK + g * 16, 16)]
            for j in range(16):
                e = g * 16 + j
                msg[e, :] = msg[e, :] * w16[j]

    def _gather(k, msg, sem):
        pltpu.async_copy(h_hbm.at[sidx.at[pl.ds(k * _CHUNK, _CHUNK)]],
                         msg, sem)

    def _gwait(msg, sem):
        pltpu.make_async_copy(
            h_hbm.at[sidx.at[pl.ds(0, _CHUNK)]], msg, sem).wait()

    def _scatter(msg, k, sem):
        return pltpu.async_copy(msg, acc.at[didx.at[k]], sem, add=True)

    _gather(0, msg0, gsem0)
    _gather(1, msg1, gsem1)

    _half = _NCHUNK // 2

    def _pair(i, carry):
        k0 = 2 * i
        k1 = k0 + 1
        _gwait(msg0, gsem0)
        _scale(msg0, k0)
        s0 = _scatter(msg0, k0, ssem0)
        _gwait(msg1, gsem1)
        _scale(msg1, k1)
        s1 = _scatter(msg1, k1, ssem1)
        s0.wait()
        _gather(k0 + 2, msg0, gsem0)
        s1.wait()

        @pl.when(i < _half - 1)
        def _():
            _gather(k1 + 2, msg1, gsem1)

        return carry

    lax.fori_loop(0, _half, _pair, 0)

    _gwait(msg0, gsem0)
    _scale(msg0, _NCHUNK - 1)
    pltpu.sync_copy(msg0, acc.at[didx.at[_NCHUNK - 1]], add=True)

    plsc.subcore_barrier()
    pltpu.sync_copy(acc.at[pl.ds(r0, _ROWS_TILE)],
                    out_hbm.at[cid, pl.ds(r0, _ROWS_TILE)])


def kernel(x, edge_index, edge_weight, kernel):
    w = kernel
    src = edge_index[0].astype(jnp.int32)
    dst = edge_index[1].astype(jnp.int32)
    ew = edge_weight.astype(jnp.float32)
    dst_r = dst.reshape(_NW, _NCHUNK, _CHUNK)
    zero = jnp.zeros((_ROWS_TILE, _F), jnp.float32)

    h = pl.pallas_call(
        _matmul_body,
        out_shape=jax.ShapeDtypeStruct((_N_NODES, _F), jnp.float32),
    )(x, w)

    mesh = plsc.VectorSubcoreMesh(core_axis_name="c", subcore_axis_name="s")
    agg_fn = functools.partial(
        pl.kernel,
        mesh=mesh,
        out_type=jax.ShapeDtypeStruct((_NC, _N_PAD, _F), jnp.float32),
        scratch_types=[
            pltpu.VMEM((_E_TILE,), jnp.int32),
            pltpu.VMEM((_NCHUNK, _CHUNK), jnp.int32),
            pltpu.VMEM((_E_TILE,), jnp.float32),
            pltpu.VMEM((_CHUNK, _F), jnp.float32),
            pltpu.VMEM((_CHUNK, _F), jnp.float32),
            pltpu.VMEM_SHARED((_N_PAD, _F), jnp.float32),
            pltpu.SemaphoreType.DMA,
            pltpu.SemaphoreType.DMA,
            pltpu.SemaphoreType.DMA,
            pltpu.SemaphoreType.DMA,
        ],
        compiler_params=pltpu.CompilerParams(use_tc_tiling_on_sc=False),
    )(_agg_body)
    parts = agg_fn(h, src, dst_r, ew, zero)

    out = pl.pallas_call(
        _softmax_body,
        grid=(_N_NODES // _SM_BLK,),
        in_specs=[pl.BlockSpec((_NC, _SM_BLK, _F), lambda i: (0, i, 0))],
        out_specs=pl.BlockSpec((_SM_BLK, _F), lambda i: (i, 0)),
        out_shape=jax.ShapeDtypeStruct((_N_NODES, _F), jnp.float32),
    )(parts)
    return out


# speedup vs baseline: 16.7401x; 1.1052x over previous
import functools

import jax
import jax.numpy as jnp
from jax import lax
from jax.experimental import pallas as pl
from jax.experimental.pallas import tpu as pltpu
from jax.experimental.pallas import tpu_sc as plsc

_N_NODES = 10000
_N_EDGES = 320000
_D = 128
_F = 16

_NC = 2
_NS = 16
_NW = _NC * _NS
_E_TILE = _N_EDGES // _NW
_CHUNK = 80
_NCHUNK = _E_TILE // _CHUNK
_ROWS_TILE = _N_NODES // _NS


def _matmul_body(x_ref, w_ref, h_ref):
    h_ref[...] = jnp.dot(x_ref[...], w_ref[...],
                         preferred_element_type=jnp.float32)


def _softmax_body(p_ref, o_ref):
    s = p_ref[0] + p_ref[1]
    m = jnp.max(s, axis=-1, keepdims=True)
    e = jnp.exp(s - m)
    o_ref[...] = e / jnp.sum(e, axis=-1, keepdims=True)


def _agg_body(h_hbm, edge_hbm, ew_hbm, zero_hbm, out_hbm,
              sidx, didx, wv, msg0, msg1, acc, gsem0, gsem1, ssem0, ssem1):
    cid = lax.axis_index("c")
    sid = lax.axis_index("s")
    wid = cid * _NS + sid
    r0 = sid * _ROWS_TILE
    e0 = wid * _E_TILE

    pltpu.sync_copy(zero_hbm, acc.at[pl.ds(r0, _ROWS_TILE)])
    pltpu.sync_copy(edge_hbm.at[0, pl.ds(e0, _E_TILE)], sidx)
    pltpu.sync_copy(edge_hbm.at[1, pl.ds(e0, _E_TILE)], didx)
    pltpu.sync_copy(ew_hbm.at[pl.ds(e0, _E_TILE)], wv)
    plsc.subcore_barrier()

    def _scale(msg, k):
        for g in range(_CHUNK // 16):
            w16 = wv[pl.ds(k * _CHUNK + g * 16, 16)]
            for j in range(16):
                e = g * 16 + j
                msg[e, :] = msg[e, :] * w16[j]

    def _gather(k, msg, sem):
        pltpu.async_copy(h_hbm.at[sidx.at[pl.ds(k * _CHUNK, _CHUNK)]],
                         msg, sem)

    def _gwait(msg, sem):
        pltpu.make_async_copy(
            h_hbm.at[sidx.at[pl.ds(0, _CHUNK)]], msg, sem).wait()

    def _scatter(msg, k, sem):
        return pltpu.async_copy(
            msg, acc.at[didx.at[pl.ds(k * _CHUNK, _CHUNK)]], sem, add=True)

    _gather(0, msg0, gsem0)
    _gather(1, msg1, gsem1)

    _half = _NCHUNK // 2

    def _pair(i, carry):
        k0 = 2 * i
        k1 = k0 + 1
        _gwait(msg0, gsem0)
        _scale(msg0, k0)
        s0 = _scatter(msg0, k0, ssem0)
        _gwait(msg1, gsem1)
        _scale(msg1, k1)
        s1 = _scatter(msg1, k1, ssem1)
        s0.wait()
        _gather(k0 + 2, msg0, gsem0)
        s1.wait()

        @pl.when(i < _half - 1)
        def _():
            _gather(k1 + 2, msg1, gsem1)

        return carry

    lax.fori_loop(0, _half, _pair, 0)

    _gwait(msg0, gsem0)
    _scale(msg0, _NCHUNK - 1)
    pltpu.sync_copy(msg0, acc.at[didx.at[pl.ds((_NCHUNK - 1) * _CHUNK,
                                               _CHUNK)]], add=True)

    plsc.subcore_barrier()
    pltpu.sync_copy(acc.at[pl.ds(r0, _ROWS_TILE)],
                    out_hbm.at[cid, pl.ds(r0, _ROWS_TILE)])


def kernel(x, edge_index, edge_weight, kernel):
    w = kernel
    edges = edge_index.astype(jnp.int32)
    ew = edge_weight.astype(jnp.float32)
    zero = jnp.zeros((_ROWS_TILE, _F), jnp.float32)

    h = pl.pallas_call(
        _matmul_body,
        out_shape=jax.ShapeDtypeStruct((_N_NODES, _F), jnp.float32),
    )(x, w)

    mesh = plsc.VectorSubcoreMesh(core_axis_name="c", subcore_axis_name="s")
    agg_fn = functools.partial(
        pl.kernel,
        mesh=mesh,
        out_type=jax.ShapeDtypeStruct((_NC, _N_NODES, _F), jnp.float32),
        scratch_types=[
            pltpu.VMEM((_E_TILE,), jnp.int32),
            pltpu.VMEM((_E_TILE,), jnp.int32),
            pltpu.VMEM((_E_TILE,), jnp.float32),
            pltpu.VMEM((_CHUNK, _F), jnp.float32),
            pltpu.VMEM((_CHUNK, _F), jnp.float32),
            pltpu.VMEM_SHARED((_N_NODES, _F), jnp.float32),
            pltpu.SemaphoreType.DMA,
            pltpu.SemaphoreType.DMA,
            pltpu.SemaphoreType.DMA,
            pltpu.SemaphoreType.DMA,
        ],
        compiler_params=pltpu.CompilerParams(use_tc_tiling_on_sc=False),
    )(_agg_body)
    parts = agg_fn(h, edges, ew, zero)

    out = pl.pallas_call(
        _softmax_body,
        out_shape=jax.ShapeDtypeStruct((_N_NODES, _F), jnp.float32),
    )(parts)
    return out


# speedup vs baseline: 20.6304x; 1.2324x over previous
import functools

import jax
import jax.numpy as jnp
from jax import lax
from jax.experimental import pallas as pl
from jax.experimental.pallas import tpu as pltpu
from jax.experimental.pallas import tpu_sc as plsc

_N_NODES = 10000
_N_EDGES = 320000
_D = 128
_F = 16

_NC = 2
_NS = 16
_NW = _NC * _NS
_E_TILE = _N_EDGES // _NW
_CHUNK = 80
_NCHUNK = _E_TILE // _CHUNK
_ROWS_TILE = _N_NODES // _NS


def _matmul_body(x_ref, w_ref, h_ref):
    h_ref[...] = jnp.dot(x_ref[...], w_ref[...],
                         preferred_element_type=jnp.float32)


def _softmax_body(p_ref, o_ref):
    s = p_ref[0] + p_ref[1]
    m = jnp.max(s, axis=-1, keepdims=True)
    e = jnp.exp(s - m)
    o_ref[...] = e / jnp.sum(e, axis=-1, keepdims=True)


def _agg_body(h_hbm, edge_hbm, ew_hbm, zero_hbm, out_hbm,
              sidx, didx, wv, msg0, msg1, msg2, msg3, acc,
              gsem0, gsem1, gsem2, gsem3, ssem0, ssem1, ssem2, ssem3):
    cid = lax.axis_index("c")
    sid = lax.axis_index("s")
    wid = cid * _NS + sid
    r0 = sid * _ROWS_TILE
    e0 = wid * _E_TILE

    pltpu.sync_copy(zero_hbm, acc.at[pl.ds(r0, _ROWS_TILE)])
    pltpu.sync_copy(edge_hbm.at[pl.ds(e0, _E_TILE)], sidx)
    pltpu.sync_copy(edge_hbm.at[pl.ds(_N_EDGES + e0, _E_TILE)], didx)
    pltpu.sync_copy(ew_hbm.at[pl.ds(e0, _E_TILE)], wv)
    plsc.subcore_barrier()

    def _scale(msg, k):
        for g in range(_CHUNK // 16):
            w16 = wv[pl.ds(k * _CHUNK + g * 16, 16)]
            for j in range(16):
                e = g * 16 + j
                msg[e, :] = msg[e, :] * w16[j]

    def _gather(k, msg, sem):
        pltpu.async_copy(h_hbm.at[sidx.at[pl.ds(k * _CHUNK, _CHUNK)]],
                         msg, sem)

    def _gwait(msg, sem):
        pltpu.make_async_copy(
            h_hbm.at[sidx.at[pl.ds(0, _CHUNK)]], msg, sem).wait()

    def _scatter(msg, k, sem):
        return pltpu.async_copy(
            msg, acc.at[didx.at[pl.ds(k * _CHUNK, _CHUNK)]], sem, add=True)

    msgs = (msg0, msg1, msg2, msg3)
    gsems = (gsem0, gsem1, gsem2, gsem3)
    ssems = (ssem0, ssem1, ssem2, ssem3)
    for b in range(4):
        _gather(b, msgs[b], gsems[b])

    _nit = (_NCHUNK - 1) // 4

    def _ring(i, carry):
        k = 4 * i
        scats = []
        for b in range(4):
            _gwait(msgs[b], gsems[b])
            _scale(msgs[b], k + b)
            scats.append(_scatter(msgs[b], k + b, ssems[b]))
        scats[0].wait()
        _gather(k + 4, msgs[0], gsems[0])
        for b in range(1, 4):
            scats[b].wait()

            @pl.when(i < _nit - 1)
            def _():
                _gather(k + 4 + b, msgs[b], gsems[b])

        return carry

    lax.fori_loop(0, _nit, _ring, 0)

    _gwait(msg0, gsem0)
    _scale(msg0, _NCHUNK - 1)
    pltpu.sync_copy(msg0, acc.at[didx.at[pl.ds((_NCHUNK - 1) * _CHUNK,
                                               _CHUNK)]], add=True)

    plsc.subcore_barrier()
    pltpu.sync_copy(acc.at[pl.ds(r0, _ROWS_TILE)],
                    out_hbm.at[cid, pl.ds(r0, _ROWS_TILE)])


def kernel(x, edge_index, edge_weight, kernel):
    w = kernel
    edges = edge_index.astype(jnp.int32).reshape(-1)
    ew = edge_weight.astype(jnp.float32)
    zero = jnp.zeros((_ROWS_TILE, _F), jnp.float32)

    h = pl.pallas_call(
        _matmul_body,
        out_shape=jax.ShapeDtypeStruct((_N_NODES, _F), jnp.float32),
    )(x, w)

    mesh = plsc.VectorSubcoreMesh(core_axis_name="c", subcore_axis_name="s")
    agg_fn = functools.partial(
        pl.kernel,
        mesh=mesh,
        out_type=jax.ShapeDtypeStruct((_NC, _N_NODES, _F), jnp.float32),
        scratch_types=[
            pltpu.VMEM((_E_TILE,), jnp.int32),
            pltpu.VMEM((_E_TILE,), jnp.int32),
            pltpu.VMEM((_E_TILE,), jnp.float32),
            pltpu.VMEM((_CHUNK, _F), jnp.float32),
            pltpu.VMEM((_CHUNK, _F), jnp.float32),
            pltpu.VMEM((_CHUNK, _F), jnp.float32),
            pltpu.VMEM((_CHUNK, _F), jnp.float32),
            pltpu.VMEM_SHARED((_N_NODES, _F), jnp.float32),
            pltpu.SemaphoreType.DMA,
            pltpu.SemaphoreType.DMA,
            pltpu.SemaphoreType.DMA,
            pltpu.SemaphoreType.DMA,
            pltpu.SemaphoreType.DMA,
            pltpu.SemaphoreType.DMA,
            pltpu.SemaphoreType.DMA,
            pltpu.SemaphoreType.DMA,
        ],
        compiler_params=pltpu.CompilerParams(use_tc_tiling_on_sc=False),
    )(_agg_body)
    parts = agg_fn(h, edges, ew, zero)

    out = pl.pallas_call(
        _softmax_body,
        out_shape=jax.ShapeDtypeStruct((_N_NODES, _F), jnp.float32),
    )(parts)
    return out
